# baseline (speedup 1.0000x reference)
import functools

import numpy as np
import jax
import jax.numpy as jnp
from jax import lax
from jax.experimental import pallas as pl
from jax.experimental.pallas import tpu as pltpu
from jax.experimental.pallas import tpu_sc as plsc

_LEVELS = [
    (64, 64, 8, -1.0, 64.0),
    (32, 32, 16, 64.0, 128.0),
    (16, 16, 32, 128.0, 256.0),
    (8, 8, 64, 256.0, 512.0),
    (4, 4, 128, 512.0, 999999.0),
]
_B = 8
_M = 100
_MPAD = 112
_NG = _MPAD // 16
_N = 5456
_NSC = 3072
_NTC = _N - _NSC
_RTC = 24
_WPB = 4
_PER_W = _NSC // _WPB
_VECS = _PER_W // 16
_NF = 11
_BIG = 9999999.0
_NONE = 127


def _build_consts():
    xs, ys, los, his, rads = [], [], [], [], []
    for (h, w, s, lo, hi) in _LEVELS:
        col = np.arange(w, dtype=np.float32) * s + s // 2
        row = np.arange(h, dtype=np.float32) * s + s // 2
        xs.append(np.tile(col, h))
        ys.append(np.repeat(row, w))
        los.append(np.full(h * w, max(lo, 0.0), np.float32))
        his.append(np.full(h * w, hi, np.float32))
        rads.append(np.full(h * w, s * 1.5, np.float32))
    x = np.concatenate(xs)
    y = np.concatenate(ys)
    lo = np.concatenate(los)
    hi = np.concatenate(his)
    rad = np.concatenate(rads)

    xs_, ys_, lo_, hi_, rad_ = (a[:_NSC] for a in (x, y, lo, hi, rad))
    xa = np.repeat(xs_.reshape(-1, 16).min(1), 16)
    xb = np.repeat(xs_.reshape(-1, 16).max(1), 16)
    ya = np.repeat(ys_.reshape(-1, 16).min(1), 16)
    yb = np.repeat(ys_.reshape(-1, 16).max(1), 16)
    t1lo = (xa - rad_ - 1.0).astype(np.float32)
    t1hi = (xb + rad_ + 1.0).astype(np.float32)
    t2lo = (ya - rad_ - 1.0).astype(np.float32)
    t2hi = (yb + rad_ + 1.0).astype(np.float32)
    lom1 = (lo_ - 1.0).astype(np.float32)
    hip1 = (hi_ + 1.0).astype(np.float32)
    fields = np.stack([xs_, ys_, lo_, hi_, rad_,
                       t1lo, t1hi, t2lo, t2hi, lom1, hip1])
    packed = np.ascontiguousarray(
        fields.reshape(_NF, -1, 16).transpose(1, 0, 2)).reshape(-1)

    pad = _RTC * 128 - _NTC
    def tpad(a, v):
        return np.concatenate(
            [a[_NSC:], np.full(pad, v, np.float32)]).reshape(_RTC, 128)
    return (packed, tpad(x, 0.0), tpad(y, 0.0), tpad(lo, 1e9),
            tpad(hi, -1e9), tpad(rad, 0.0))


(_LOCPACK, _XT, _YT, _LOT, _HIT, _RADT) = _build_consts()


def _sqrt16(x):
    i = lax.bitcast_convert_type(x, jnp.int32)
    i = 0x5F3759DF - (i >> 1)
    y = lax.bitcast_convert_type(i, jnp.float32)
    for _ in range(3):
        y = y * (1.5 - 0.5 * x * y * y)
    return x * y


def _shuf(vv, perm):
    return jnp.take_along_axis(vv, perm, axis=0)


def _hmin(vv, iota):
    for k in (1, 2, 4, 8):
        vv = jnp.minimum(vv, _shuf(vv, iota ^ k))
    return vv


def _hsum(vv, iota):
    for k in (1, 2, 4, 8):
        vv = vv + _shuf(vv, iota ^ k)
    return vv


def _sc_body(bx_hbm, cls_hbm, loc_hbm,
             cls_out, cnt_out, l_out, t_out, r_out, b_out,
             bxb, clsb, locv,
             ocls, ocnt, ol, ot, orr, ob, dsem):
    wid = lax.axis_index("s") * 2 + lax.axis_index("c")
    b = wid // _WPB
    q = wid % _WPB
    off = q * _PER_W

    c1 = pltpu.async_copy(bx_hbm.at[pl.ds(b * (_NG * 64), _NG * 64)], bxb, dsem)
    c2 = pltpu.async_copy(cls_hbm.at[pl.ds(b * _MPAD, _MPAD)], clsb, dsem)
    c3 = pltpu.async_copy(
        loc_hbm.at[pl.ds(q * (_VECS * _NF * 16), _VECS * _NF * 16)], locv, dsem)
    c1.wait()
    c2.wait()
    c3.wait()

    iota = lax.iota(jnp.int32, 16)

    gscan = []
    for g in range(_NG):
        gb = g * 64
        x1g = bxb[pl.ds(gb, 16)]
        y1g = bxb[pl.ds(gb + 16, 16)]
        x2g = bxb[pl.ds(gb + 32, 16)]
        y2g = bxb[pl.ds(gb + 48, 16)]
        w = x2g - x1g
        h = y2g - y1g
        gscan.append(((x1g + x2g) / 2, (y1g + y2g) / 2,
                      jnp.minimum(w, h) * 0.5, jnp.maximum(w, h) * 0.5))

    def one_vec(v):
        base = v * (_NF * 16)
        x = locv[pl.ds(base, 16)]
        y = locv[pl.ds(base + 16, 16)]
        lo = locv[pl.ds(base + 32, 16)]
        hi = locv[pl.ds(base + 48, 16)]
        rad = locv[pl.ds(base + 64, 16)]
        t1lo = locv[pl.ds(base + 80, 16)]
        t1hi = locv[pl.ds(base + 96, 16)]
        t2lo = locv[pl.ds(base + 112, 16)]
        t2hi = locv[pl.ds(base + 128, 16)]
        lom1 = locv[pl.ds(base + 144, 16)]
        hip1 = locv[pl.ds(base + 160, 16)]

        gvals = []
        tcount = jnp.zeros((16,), jnp.int32)
        for g, (cxg, cyg, mw, Mw) in enumerate(gscan):
            t = ((cxg > t1lo) & (cxg < t1hi) & (cyg > t2lo) & (cyg < t2hi)
                 & (mw > lom1) & (Mw < hip1))
            gvals.append(jnp.where(t, iota + g * 16, _NONE))
            tcount = tcount + jnp.where(t, 1, 0)
        gmin = gvals[0]
        for gv in gvals[1:]:
            gmin = jnp.minimum(gmin, gv)
        jm0 = _hmin(gmin, iota)
        cg = _hsum(tcount, iota)[0]

        def w_body(i, st):
            jm = st[0]
            gv = list(st[1:1 + _NG])
            ba, bl, bt, br, bb, bc = st[1 + _NG:]
            js = jm[0]
            gb = (js >> 4) * 64
            lane = jm & 15
            x1 = _shuf(bxb[pl.ds(gb, 16)], lane)
            y1 = _shuf(bxb[pl.ds(gb + 16, 16)], lane)
            x2 = _shuf(bxb[pl.ds(gb + 32, 16)], lane)
            y2 = _shuf(bxb[pl.ds(gb + 48, 16)], lane)
            cj = _shuf(clsb[pl.ds((js >> 4) * 16, 16)], lane)
            l = x - x1
            t_ = y - y1
            r = x2 - x
            bo = y2 - y
            area = (l + r) * (t_ + bo)
            omin = jnp.minimum(jnp.minimum(l, t_), jnp.minimum(r, bo))
            omax = jnp.maximum(jnp.maximum(l, t_), jnp.maximum(r, bo))
            cxs = (x1 + x2) / 2
            cys = (y1 + y2) / 2
            cmax = jnp.maximum(jnp.abs(x - cxs), jnp.abs(y - cys))
            mask = (omin > lo) & (omax <= hi) & (cmax < rad)
            take = mask & (area < ba)
            nb = (jnp.where(take, area, ba),
                  jnp.where(take, l, bl),
                  jnp.where(take, t_, bt),
                  jnp.where(take, r, br),
                  jnp.where(take, bo, bb),
                  jnp.where(take, cj, bc))
            gv = [jnp.where(g == jm, _NONE, g) for g in gv]
            gm = gv[0]
            for g in gv[1:]:
                gm = jnp.minimum(gm, g)
            return (_hmin(gm, iota),) + tuple(gv) + nb

        zero = jnp.zeros((16,), jnp.float32)
        init = ((jm0,) + tuple(gvals)
                + (jnp.full((16,), _BIG, jnp.float32), zero, zero, zero, zero,
                   jnp.zeros((16,), jnp.int32)))
        st = lax.fori_loop(0, cg, w_body, init)
        ba, bl, bt, br, bb, bc = st[1 + _NG:]

        pos = ba < _BIG
        lr_min = jnp.minimum(bl, br)
        lr_max = jnp.maximum(bl, br)
        tb_min = jnp.minimum(bt, bb)
        tb_max = jnp.maximum(bt, bb)
        ratio = lr_min * tb_min / (lr_max * tb_max + 1e-10)
        cnt = _sqrt16(jnp.where(pos, ratio, 1.0))
        neg1 = jnp.full((16,), -1.0, jnp.float32)
        bs = pl.ds(v * 16, 16)
        ocls[bs] = jnp.where(pos, bc, 0)
        ocnt[bs] = jnp.where(pos, cnt, neg1)
        ol[bs] = jnp.where(pos, bl, neg1)
        ot[bs] = jnp.where(pos, bt, neg1)
        orr[bs] = jnp.where(pos, br, neg1)
        ob[bs] = jnp.where(pos, bb, neg1)

    def vec_body(u, carry):
        one_vec(u * 2)
        one_vec(u * 2 + 1)
        return carry

    lax.fori_loop(0, _VECS // 2, vec_body, 0)

    oflat = b * _NSC + off
    pltpu.sync_copy(ocls, cls_out.at[pl.ds(oflat, _PER_W)])
    pltpu.sync_copy(ocnt, cnt_out.at[pl.ds(oflat, _PER_W)])
    pltpu.sync_copy(ol, l_out.at[pl.ds(oflat, _PER_W)])
    pltpu.sync_copy(ot, t_out.at[pl.ds(oflat, _PER_W)])
    pltpu.sync_copy(orr, r_out.at[pl.ds(oflat, _PER_W)])
    pltpu.sync_copy(ob, b_out.at[pl.ds(oflat, _PER_W)])


@functools.cache
def _build_encode():
  f32 = jnp.float32
  i32 = jnp.int32
  return functools.partial(
    pl.kernel,
    mesh=plsc.VectorSubcoreMesh(core_axis_name="c", subcore_axis_name="s"),
    out_type=[
        jax.ShapeDtypeStruct((_B * _NSC,), i32),
        jax.ShapeDtypeStruct((_B * _NSC,), f32),
        jax.ShapeDtypeStruct((_B * _NSC,), f32),
        jax.ShapeDtypeStruct((_B * _NSC,), f32),
        jax.ShapeDtypeStruct((_B * _NSC,), f32),
        jax.ShapeDtypeStruct((_B * _NSC,), f32),
    ],
    scratch_types=[
        pltpu.VMEM((_NG * 64,), f32),
        pltpu.VMEM((_MPAD,), i32),
        pltpu.VMEM((_VECS * _NF * 16,), f32),
        pltpu.VMEM((_PER_W,), i32),
        pltpu.VMEM((_PER_W,), f32),
        pltpu.VMEM((_PER_W,), f32),
        pltpu.VMEM((_PER_W,), f32),
        pltpu.VMEM((_PER_W,), f32),
        pltpu.VMEM((_PER_W,), f32),
        pltpu.SemaphoreType.DMA,
    ],
  )(_sc_body)


def _tc_body(boxes_s, cls_s, xr, yr, lor, hir, radr,
             ocls, ocnt, ol, ot, orr, ob):
    x = xr[...]
    y = yr[...]
    lo = lor[...]
    hi = hir[...]
    rad = radr[...]

    def box_body(j, st):
        ba, bl, bt, br, bb, bc = st
        x1 = boxes_s[0, j, 0]
        y1 = boxes_s[0, j, 1]
        x2 = boxes_s[0, j, 2]
        y2 = boxes_s[0, j, 3]
        cj = cls_s[0, 0, j]
        l = x - x1
        t_ = y - y1
        r = x2 - x
        bo = y2 - y
        area = (l + r) * (t_ + bo)
        omin = jnp.minimum(jnp.minimum(l, t_), jnp.minimum(r, bo))
        omax = jnp.maximum(jnp.maximum(l, t_), jnp.maximum(r, bo))
        cx = (x1 + x2) / 2
        cy = (y1 + y2) / 2
        cmax = jnp.maximum(jnp.abs(x - cx), jnp.abs(y - cy))
        mask = (omin > lo) & (omax <= hi) & (cmax < rad)
        take = mask & (area < ba)
        return (jnp.where(take, area, ba),
                jnp.where(take, l, bl),
                jnp.where(take, t_, bt),
                jnp.where(take, r, br),
                jnp.where(take, bo, bb),
                jnp.where(take, cj, bc))

    zero = jnp.zeros((_RTC, 128), jnp.float32)
    init = (jnp.full((_RTC, 128), _BIG, jnp.float32), zero, zero, zero, zero,
            jnp.zeros((_RTC, 128), jnp.int32))
    ba, bl, bt, br, bb, bc = lax.fori_loop(0, _M, box_body, init)

    pos = ba < _BIG
    lr_min = jnp.minimum(bl, br)
    lr_max = jnp.maximum(bl, br)
    tb_min = jnp.minimum(bt, bb)
    tb_max = jnp.maximum(bt, bb)
    ratio = lr_min * tb_min / (lr_max * tb_max + 1e-10)
    cnt = jnp.sqrt(jnp.where(pos, ratio, 1.0))
    ocls[0] = jnp.where(pos, bc, 0)
    ocnt[0] = jnp.where(pos, cnt, -1.0)
    ol[0] = jnp.where(pos, bl, -1.0)
    ot[0] = jnp.where(pos, bt, -1.0)
    orr[0] = jnp.where(pos, br, -1.0)
    ob[0] = jnp.where(pos, bb, -1.0)


@functools.cache
def _build_tc():
  f32 = jnp.float32
  i32 = jnp.int32
  loc_spec = pl.BlockSpec((_RTC, 128), lambda b: (0, 0))
  out_spec = pl.BlockSpec((1, _RTC, 128), lambda b: (b, 0, 0))
  return pl.pallas_call(
      _tc_body,
      grid=(_B,),
      in_specs=[
          pl.BlockSpec((1, _M, 4), lambda b: (b, 0, 0),
                       memory_space=pltpu.SMEM),
          pl.BlockSpec((1, 1, _M), lambda b: (b, 0, 0),
                       memory_space=pltpu.SMEM),
          loc_spec, loc_spec, loc_spec, loc_spec, loc_spec,
      ],
      out_specs=[out_spec] * 6,
      out_shape=[
          jax.ShapeDtypeStruct((_B, _RTC, 128), i32),
          jax.ShapeDtypeStruct((_B, _RTC, 128), f32),
          jax.ShapeDtypeStruct((_B, _RTC, 128), f32),
          jax.ShapeDtypeStruct((_B, _RTC, 128), f32),
          jax.ShapeDtypeStruct((_B, _RTC, 128), f32),
          jax.ShapeDtypeStruct((_B, _RTC, 128), f32),
      ],
  )


def kernel(cls_p3, cnt_p3, reg_p3, cls_p4, cnt_p4, reg_p4, cls_p5, cnt_p5,
           reg_p5, cls_p6, cnt_p6, reg_p6, cls_p7, cnt_p7, reg_p7,
           gt_boxes, classes):
    del cls_p3, cnt_p3, reg_p3, cls_p4, cnt_p4, reg_p4, cls_p5, cnt_p5
    del reg_p5, cls_p6, cnt_p6, reg_p6, cls_p7, cnt_p7, reg_p7
    gt = gt_boxes.astype(jnp.float32)
    gtp = jnp.pad(gt, ((0, 0), (0, _MPAD - _M), (0, 0)), constant_values=1e9)
    bx = gtp.reshape(_B, _NG, 16, 4).transpose(0, 1, 3, 2).reshape(-1)
    clsc = jnp.pad(classes.astype(jnp.int32),
                   ((0, 0), (0, _MPAD - _M))).reshape(-1)
    cls_f, cnt_f, l_f, t_f, r_f, b_f = _build_encode()(
        bx, clsc, jnp.asarray(_LOCPACK))
    tcls, tcnt, tl, tt, tr, tb = _build_tc()(
        gt, classes.astype(jnp.int32)[:, None, :], jnp.asarray(_XT), jnp.asarray(_YT),
        jnp.asarray(_LOT), jnp.asarray(_HIT), jnp.asarray(_RADT))

    def comb(sc, tc):
        return jnp.concatenate(
            [sc.reshape(_B, _NSC), tc.reshape(_B, _RTC * 128)[:, :_NTC]],
            axis=1)

    cls_c = comb(cls_f, tcls)
    cnt_c = comb(cnt_f, tcnt)
    l_c = comb(l_f, tl)
    t_c = comb(t_f, tt)
    r_c = comb(r_f, tr)
    b_c = comb(b_f, tb)
    cls_t = cls_c[:, :, None]
    cnt_t = cnt_c[:, :, None]
    reg_t = jnp.stack([l_c, t_c, r_c, b_c], axis=-1)
    return cls_t, cnt_t, reg_t
